# baseline (speedup 1.0000x reference)
import jax
import jax.numpy as jnp
from jax import lax
from jax.experimental import pallas as pl
from jax.experimental.pallas import tpu as pltpu
from jax.experimental.pallas import tpu_sc as plsc

_N = 10000
_NP = 10240
_E = 320000
_D = 128
_DP = 144
_NC = 2
_NS = 16
_NW = _NC * _NS
_PER_W = _E // _NW
_CH = 80
_NCHUNK = _PER_W // _CH
_ZPS = _NP // _CH // _NS


def _sc_body(x_hbm, src_hbm, dst_hbm, acc_out,
             acc_sh, isrc0, idst0, isrc1, idst1, rows0, rows1,
             sem_g0, sem_g1, sem_s0, sem_s1, sem_d0, sem_d1):
    c = lax.axis_index("c")
    s = lax.axis_index("s")
    wid = c * _NS + s
    zeros16 = jnp.zeros((16,), jnp.float32)

    def _zero_rows(i, carry):
        rows0[i // (_DP // 16), pl.ds((i % (_DP // 16)) * 16, 16)] = zeros16
        return carry
    lax.fori_loop(0, _CH * (_DP // 16), _zero_rows, 0)

    def _zero_acc(t, carry):
        k = s + _NS * t
        pltpu.sync_copy(rows0, acc_sh.at[pl.ds(k * _CH, _CH)])
        return carry
    lax.fori_loop(0, _ZPS, _zero_acc, 0)
    plsc.subcore_barrier()

    pltpu.sync_copy(src_hbm.at[wid, 0], isrc0)
    pltpu.sync_copy(dst_hbm.at[wid, 0], idst0)
    pltpu.async_copy(src_hbm.at[wid, 1], isrc1, sem_s1)
    pltpu.async_copy(dst_hbm.at[wid, 1], idst1, sem_d1)
    pltpu.async_copy(x_hbm.at[isrc0], rows0, sem_g0)

    def _half(k, isrc_a, idst_a, rows_a, sem_ga, sem_sa, sem_da,
              isrc_b, idst_b, rows_b, sem_gb, sem_sb, sem_db):
        @pl.when(k + 1 < _NCHUNK)
        def _():
            pltpu.make_async_copy(src_hbm.at[wid, k + 1], isrc_b, sem_sb).wait()
            pltpu.make_async_copy(dst_hbm.at[wid, k + 1], idst_b, sem_db).wait()
        pltpu.make_async_copy(x_hbm.at[isrc_a], rows_a, sem_ga).wait()

        @pl.when(k + 1 < _NCHUNK)
        def _():
            pltpu.async_copy(x_hbm.at[isrc_b], rows_b, sem_gb)
        pltpu.sync_copy(rows_a, acc_sh.at[idst_a], add=True)

        @pl.when(k + 2 < _NCHUNK)
        def _():
            pltpu.async_copy(src_hbm.at[wid, k + 2], isrc_a, sem_sa)
            pltpu.async_copy(dst_hbm.at[wid, k + 2], idst_a, sem_da)

    def _step(t, carry):
        k = 2 * t
        _half(k, isrc0, idst0, rows0, sem_g0, sem_s0, sem_d0,
              isrc1, idst1, rows1, sem_g1, sem_s1, sem_d1)

        @pl.when(k + 1 < _NCHUNK)
        def _():
            _half(k + 1, isrc1, idst1, rows1, sem_g1, sem_s1, sem_d1,
                  isrc0, idst0, rows0, sem_g0, sem_s0, sem_d0)
        return carry
    lax.fori_loop(0, (_NCHUNK + 1) // 2, _step, 0)
    plsc.subcore_barrier()

    def _copy_out(t, carry):
        r0 = (s + _NS * t) * _CH
        pltpu.sync_copy(acc_sh.at[pl.ds(r0, _CH)], acc_out.at[c, pl.ds(r0, _CH)])
        return carry
    lax.fori_loop(0, _ZPS, _copy_out, 0)


def _sc_aggregate(xp, src, dst):
    mesh = plsc.VectorSubcoreMesh(core_axis_name="c", subcore_axis_name="s")
    fn = pl.kernel(
        _sc_body,
        out_type=jax.ShapeDtypeStruct((_NC, _NP, _DP), jnp.float32),
        mesh=mesh,
        compiler_params=pltpu.CompilerParams(use_tc_tiling_on_sc=False),
        scratch_types=[
            pltpu.VMEM_SHARED((_NP, _DP), jnp.float32),
            pltpu.VMEM((_CH,), jnp.int32),
            pltpu.VMEM((_CH,), jnp.int32),
            pltpu.VMEM((_CH,), jnp.int32),
            pltpu.VMEM((_CH,), jnp.int32),
            pltpu.VMEM((_CH, _DP), jnp.float32),
            pltpu.VMEM((_CH, _DP), jnp.float32),
            pltpu.SemaphoreType.DMA,
            pltpu.SemaphoreType.DMA,
            pltpu.SemaphoreType.DMA,
            pltpu.SemaphoreType.DMA,
            pltpu.SemaphoreType.DMA,
            pltpu.SemaphoreType.DMA,
        ],
    )
    return fn(xp, src, dst)


def _tc_body(acc_ref, w_ref, b_ref, o_ref):
    s = acc_ref[0] + acc_ref[1]
    y = jnp.dot(s[:, :_D], w_ref[...], preferred_element_type=jnp.float32)
    cnt = jnp.maximum(s[:, _D:_D + 1], 1.0)
    o_ref[...] = jnp.maximum(y / cnt + b_ref[...], 0.0)


_BR = 512


def _tc_finish(acc, W, b2):
    return pl.pallas_call(
        _tc_body,
        grid=(_NP // _BR,),
        in_specs=[
            pl.BlockSpec((_NC, _BR, _DP), lambda i: (0, i, 0)),
            pl.BlockSpec((_D, _D), lambda i: (0, 0)),
            pl.BlockSpec((1, _D), lambda i: (0, 0)),
        ],
        out_specs=pl.BlockSpec((_BR, _D), lambda i: (i, 0)),
        out_shape=jax.ShapeDtypeStruct((_NP, _D), jnp.float32),
    )(acc, W, b2)


def kernel(x, edge_index, W, b):
    xp = jnp.concatenate(
        [x, jnp.ones((_N, 1), jnp.float32), jnp.zeros((_N, _DP - _D - 1), jnp.float32)],
        axis=1)
    e3 = edge_index.reshape(2, _NW, _NCHUNK, _CH)
    acc = _sc_aggregate(xp, e3[0], e3[1])
    out = _tc_finish(acc, W, b.reshape(1, _D))
    return out[:_N]


# speedup vs baseline: 1.2110x; 1.2110x over previous
import jax
import jax.numpy as jnp
from jax import lax
from jax.experimental import pallas as pl
from jax.experimental.pallas import tpu as pltpu
from jax.experimental.pallas import tpu_sc as plsc

_N = 10000
_NP = 10240
_E = 320000
_D = 128
_NC = 2
_NS = 16
_NW = _NC * _NS
_PER_W = _E // _NW
_CH = 80
_NCHUNK = _PER_W // _CH
_ZPS = _NP // _CH // _NS


def _sc_body(x_hbm, src_hbm, dst_hbm, acc_out, cnt_out,
             acc_sh, cnt_sh, isrc0, idst0, isrc1, idst1, rows0, rows1,
             ones_v, zc_v, sem_g0, sem_g1, sem_s0, sem_s1, sem_d0, sem_d1):
    c = lax.axis_index("c")
    s = lax.axis_index("s")
    wid = c * _NS + s
    zeros16 = jnp.zeros((16,), jnp.float32)
    ones16 = jnp.ones((16,), jnp.float32)

    for i in range(_CH // 16):
        ones_v[pl.ds(i * 16, 16)] = ones16
        zc_v[pl.ds(i * 16, 16)] = zeros16

    def _zero_rows(i, carry):
        rows0[i // 8, pl.ds((i % 8) * 16, 16)] = zeros16
        return carry
    lax.fori_loop(0, _CH * (_D // 16), _zero_rows, 0)

    def _zero_acc(t, carry):
        k = s + _NS * t
        pltpu.sync_copy(rows0, acc_sh.at[pl.ds(k * _CH, _CH)])
        pltpu.sync_copy(zc_v, cnt_sh.at[pl.ds(k * _CH, _CH)])
        return carry
    lax.fori_loop(0, _ZPS, _zero_acc, 0)
    plsc.subcore_barrier()

    pltpu.sync_copy(src_hbm.at[wid, 0], isrc0)
    pltpu.sync_copy(dst_hbm.at[wid, 0], idst0)
    pltpu.async_copy(src_hbm.at[wid, 1], isrc1, sem_s1)
    pltpu.async_copy(dst_hbm.at[wid, 1], idst1, sem_d1)
    pltpu.async_copy(x_hbm.at[isrc0], rows0, sem_g0)

    def _half(k, isrc_a, idst_a, rows_a, sem_ga, sem_sa, sem_da,
              isrc_b, idst_b, rows_b, sem_gb, sem_sb, sem_db):
        @pl.when(k + 1 < _NCHUNK)
        def _():
            pltpu.make_async_copy(src_hbm.at[wid, k + 1], isrc_b, sem_sb).wait()
            pltpu.make_async_copy(dst_hbm.at[wid, k + 1], idst_b, sem_db).wait()
        pltpu.make_async_copy(x_hbm.at[isrc_a], rows_a, sem_ga).wait()

        @pl.when(k + 1 < _NCHUNK)
        def _():
            pltpu.async_copy(x_hbm.at[isrc_b], rows_b, sem_gb)
        pltpu.sync_copy(rows_a, acc_sh.at[idst_a], add=True)
        pltpu.sync_copy(ones_v, cnt_sh.at[idst_a], add=True)

        @pl.when(k + 2 < _NCHUNK)
        def _():
            pltpu.async_copy(src_hbm.at[wid, k + 2], isrc_a, sem_sa)
            pltpu.async_copy(dst_hbm.at[wid, k + 2], idst_a, sem_da)

    def _step(t, carry):
        k = 2 * t
        _half(k, isrc0, idst0, rows0, sem_g0, sem_s0, sem_d0,
              isrc1, idst1, rows1, sem_g1, sem_s1, sem_d1)

        @pl.when(k + 1 < _NCHUNK)
        def _():
            _half(k + 1, isrc1, idst1, rows1, sem_g1, sem_s1, sem_d1,
                  isrc0, idst0, rows0, sem_g0, sem_s0, sem_d0)
        return carry
    lax.fori_loop(0, (_NCHUNK + 1) // 2, _step, 0)
    plsc.subcore_barrier()

    def _copy_out(t, carry):
        r0 = (s + _NS * t) * _CH
        pltpu.sync_copy(acc_sh.at[pl.ds(r0, _CH)], acc_out.at[c, pl.ds(r0, _CH)])
        return carry
    lax.fori_loop(0, _ZPS, _copy_out, 0)
    pltpu.sync_copy(cnt_sh.at[pl.ds(s * (_NP // _NS), _NP // _NS)],
                    cnt_out.at[c, pl.ds(s * (_NP // _NS), _NP // _NS)])


def _sc_aggregate(x, src, dst):
    mesh = plsc.VectorSubcoreMesh(core_axis_name="c", subcore_axis_name="s")
    fn = pl.kernel(
        _sc_body,
        out_type=[
            jax.ShapeDtypeStruct((_NC, _NP, _D), jnp.float32),
            jax.ShapeDtypeStruct((_NC, _NP), jnp.float32),
        ],
        mesh=mesh,
        scratch_types=[
            pltpu.VMEM_SHARED((_NP, _D), jnp.float32),
            pltpu.VMEM_SHARED((_NP,), jnp.float32),
            pltpu.VMEM((_CH,), jnp.int32),
            pltpu.VMEM((_CH,), jnp.int32),
            pltpu.VMEM((_CH,), jnp.int32),
            pltpu.VMEM((_CH,), jnp.int32),
            pltpu.VMEM((_CH, _D), jnp.float32),
            pltpu.VMEM((_CH, _D), jnp.float32),
            pltpu.VMEM((_CH,), jnp.float32),
            pltpu.VMEM((_CH,), jnp.float32),
            pltpu.SemaphoreType.DMA,
            pltpu.SemaphoreType.DMA,
            pltpu.SemaphoreType.DMA,
            pltpu.SemaphoreType.DMA,
            pltpu.SemaphoreType.DMA,
            pltpu.SemaphoreType.DMA,
        ],
    )
    return fn(x, src, dst)


def _tc_body(acc_ref, cnt_ref, w_ref, b_ref, o_ref):
    s = acc_ref[0] + acc_ref[1]
    ones_col = jnp.ones((_NC, 1), jnp.float32)
    cnt_col = lax.dot_general(cnt_ref[...], ones_col,
                              (((0,), (0,)), ((), ())),
                              preferred_element_type=jnp.float32)
    cnt_col = jnp.maximum(cnt_col, 1.0)
    y = jnp.dot(s, w_ref[...], preferred_element_type=jnp.float32)
    o_ref[...] = jnp.maximum(y / cnt_col + b_ref[...], 0.0)


_BR = 512


def _tc_finish(acc, cnt, W, b2):
    return pl.pallas_call(
        _tc_body,
        grid=(_NP // _BR,),
        in_specs=[
            pl.BlockSpec((_NC, _BR, _D), lambda i: (0, i, 0)),
            pl.BlockSpec((_NC, _BR), lambda i: (0, i)),
            pl.BlockSpec((_D, _D), lambda i: (0, 0)),
            pl.BlockSpec((1, _D), lambda i: (0, 0)),
        ],
        out_specs=pl.BlockSpec((_BR, _D), lambda i: (i, 0)),
        out_shape=jax.ShapeDtypeStruct((_NP, _D), jnp.float32),
    )(acc, cnt, W, b2)


def kernel(x, edge_index, W, b):
    e3 = edge_index.reshape(2, _NW, _NCHUNK, _CH)
    acc, cnt = _sc_aggregate(x, e3[0], e3[1])
    out = _tc_finish(acc, cnt, W, b.reshape(1, _D))
    return out[:_N]


# speedup vs baseline: 1.6131x; 1.3320x over previous
import jax
import jax.numpy as jnp
from jax import lax
from jax.experimental import pallas as pl
from jax.experimental.pallas import tpu as pltpu
from jax.experimental.pallas import tpu_sc as plsc

_N = 10000
_NP = 10240
_E = 320000
_D = 128
_NC = 2
_NS = 16
_NW = _NC * _NS
_CH = 128
_NCHK = _E // _CH
_NKB = _NCHK // _NW
_NEXTRA = _NCHK - _NKB * _NW
_ZPS = _NP // _CH // _NS


def _sc_body(x_hbm, edge_hbm, acc_out, cnt_out,
             acc_sh, cnt_sh, idx0, idx1, rows0, rows1,
             ones_v, zc_v, sem_i0, sem_i1, sem_g0, sem_g1, sem_c0, sem_c1):
    c = lax.axis_index("c")
    s = lax.axis_index("s")
    wid = c * _NS + s
    nk = jnp.where(wid < _NEXTRA, _NKB + 1, _NKB)
    zeros16 = jnp.zeros((16,), jnp.float32)
    ones16 = jnp.ones((16,), jnp.float32)

    for i in range(_CH // 16):
        ones_v[pl.ds(i * 16, 16)] = ones16
        zc_v[pl.ds(i * 16, 16)] = zeros16

    def _zero_rows(i, carry):
        rows0[i // 8, pl.ds((i % 8) * 16, 16)] = zeros16
        return carry
    lax.fori_loop(0, _CH * (_D // 16), _zero_rows, 0)

    def _zero_acc(t, carry):
        k = s + _NS * t
        pltpu.sync_copy(rows0, acc_sh.at[pl.ds(k * _CH, _CH)])
        pltpu.sync_copy(zc_v, cnt_sh.at[pl.ds(k * _CH, _CH)])
        return carry
    lax.fori_loop(0, _ZPS, _zero_acc, 0)
    plsc.subcore_barrier()

    pltpu.sync_copy(edge_hbm.at[:, pl.ds(wid * _CH, _CH)], idx0)
    pltpu.async_copy(edge_hbm.at[:, pl.ds((wid + _NW) * _CH, _CH)], idx1,
                     sem_i1)
    pltpu.async_copy(x_hbm.at[idx0.at[0]], rows0, sem_g0)

    def _half(k, idx_a, rows_a, sem_ia, sem_ga, sem_ca,
              idx_b, rows_b, sem_ib, sem_gb, sem_cb):
        @pl.when(k + 1 < nk)
        def _():
            pltpu.make_async_copy(edge_hbm.at[:, pl.ds(0, _CH)], idx_b,
                                  sem_ib).wait()
        pltpu.make_async_copy(x_hbm.at[idx_a.at[0]], rows_a, sem_ga).wait()

        @pl.when(k + 1 < nk)
        def _():
            pltpu.async_copy(x_hbm.at[idx_b.at[0]], rows_b, sem_gb)
        pltpu.async_copy(ones_v, cnt_sh.at[idx_a.at[1]], sem_ca, add=True)
        pltpu.sync_copy(rows_a, acc_sh.at[idx_a.at[1]], add=True)
        pltpu.make_async_copy(ones_v, cnt_sh.at[idx_a.at[1]], sem_ca).wait()

        @pl.when(k + 2 < nk)
        def _():
            pltpu.async_copy(
                edge_hbm.at[:, pl.ds((wid + _NW * (k + 2)) * _CH, _CH)],
                idx_a, sem_ia)

    def _step(t, carry):
        k = 2 * t
        _half(k, idx0, rows0, sem_i0, sem_g0, sem_c0,
              idx1, rows1, sem_i1, sem_g1, sem_c1)

        @pl.when(k + 1 < nk)
        def _():
            _half(k + 1, idx1, rows1, sem_i1, sem_g1, sem_c1,
                  idx0, rows0, sem_i0, sem_g0, sem_c0)
        return carry
    lax.fori_loop(0, (nk + 1) // 2, _step, 0)
    plsc.subcore_barrier()

    def _copy_out(t, carry):
        r0 = (s + _NS * t) * _CH
        pltpu.sync_copy(acc_sh.at[pl.ds(r0, _CH)], acc_out.at[c, pl.ds(r0, _CH)])
        return carry
    lax.fori_loop(0, _ZPS, _copy_out, 0)
    pltpu.sync_copy(cnt_sh.at[pl.ds(s * (_NP // _NS), _NP // _NS)],
                    cnt_out.at[c, pl.ds(s * (_NP // _NS), _NP // _NS)])


def _sc_aggregate(x, edge_index):
    mesh = plsc.VectorSubcoreMesh(core_axis_name="c", subcore_axis_name="s")
    fn = pl.kernel(
        _sc_body,
        out_type=[
            jax.ShapeDtypeStruct((_NC, _NP, _D), jnp.float32),
            jax.ShapeDtypeStruct((_NC, _NP), jnp.float32),
        ],
        mesh=mesh,
        scratch_types=[
            pltpu.VMEM_SHARED((_NP, _D), jnp.float32),
            pltpu.VMEM_SHARED((_NP,), jnp.float32),
            pltpu.VMEM((2, _CH), jnp.int32),
            pltpu.VMEM((2, _CH), jnp.int32),
            pltpu.VMEM((_CH, _D), jnp.float32),
            pltpu.VMEM((_CH, _D), jnp.float32),
            pltpu.VMEM((_CH,), jnp.float32),
            pltpu.VMEM((_CH,), jnp.float32),
            pltpu.SemaphoreType.DMA,
            pltpu.SemaphoreType.DMA,
            pltpu.SemaphoreType.DMA,
            pltpu.SemaphoreType.DMA,
            pltpu.SemaphoreType.DMA,
            pltpu.SemaphoreType.DMA,
        ],
    )
    return fn(x, edge_index)


def _inv_body(cnt_ref, inv_ref):
    ones_col = jnp.ones((_NC, 1), jnp.float32)
    cnt_col = lax.dot_general(cnt_ref[...], ones_col,
                              (((0,), (0,)), ((), ())),
                              preferred_element_type=jnp.float32)
    inv_ref[...] = 1.0 / jnp.maximum(cnt_col, 1.0)


def _tc_body(acc_ref, inv_ref, w_ref, b_ref, o_ref):
    s = acc_ref[0] + acc_ref[1]
    y = jnp.dot(s, w_ref[...], preferred_element_type=jnp.float32)
    o_ref[...] = jnp.maximum(y * inv_ref[...] + b_ref[...], 0.0)


_BR = 1000


def _tc_finish(acc, cnt, W, b2):
    inv = pl.pallas_call(
        _inv_body,
        out_shape=jax.ShapeDtypeStruct((_NP, 1), jnp.float32),
    )(cnt)
    return pl.pallas_call(
        _tc_body,
        grid=(_N // _BR,),
        in_specs=[
            pl.BlockSpec((_NC, _BR, _D), lambda i: (0, i, 0)),
            pl.BlockSpec((_BR, 1), lambda i: (i, 0)),
            pl.BlockSpec((_D, _D), lambda i: (0, 0)),
            pl.BlockSpec((1, _D), lambda i: (0, 0)),
        ],
        out_specs=pl.BlockSpec((_BR, _D), lambda i: (i, 0)),
        out_shape=jax.ShapeDtypeStruct((_N, _D), jnp.float32),
    )(acc, inv, W, b2)


def kernel(x, edge_index, W, b):
    acc, cnt = _sc_aggregate(x, edge_index)
    return _tc_finish(acc, cnt, W, b.reshape(1, _D))
